# baseline (speedup 1.0000x reference)
import jax
import jax.numpy as jnp
from jax import lax
from jax.experimental import pallas as pl
from jax.experimental.pallas import tpu as pltpu

_PLACEHOLDER = 9
_LR_MUL = 0.1


def _mlp_body(tok_ref, face_ref, w0_ref, b0_ref, w1_ref, b1_ref, cm_ref, cs_ref,
              tie0_ref, tie1_ref, rloc_ref):
    x = face_ref[...]
    nrm = jnp.sqrt(jnp.sum(x * x, axis=1, keepdims=True))
    x = x / jnp.maximum(nrm, 1e-12)
    h = lax.dot_general(x, w0_ref[...], (((1,), (1,)), ((), ())),
                        preferred_element_type=jnp.float32)
    h = h * _LR_MUL + b0_ref[...] * _LR_MUL
    h = jnp.where(h >= 0, h, 0.2 * h)
    r = lax.dot_general(h, w1_ref[...], (((1,), (1,)), ((), ())),
                        preferred_element_type=jnp.float32)
    r = r * _LR_MUL + b1_ref[...] * _LR_MUL
    r = jnp.where(r >= 0, r, 0.2 * r)
    d = cm_ref.shape[1]
    tie0_ref[...] = cm_ref[0:1, :] + r[:, :d] * cs_ref[0:1, :]
    tie1_ref[...] = cm_ref[1:2, :] + r[:, d:] * cs_ref[1:2, :]
    tok = tok_ref[...]
    bm, n_seq = tok.shape
    iota = lax.broadcasted_iota(jnp.int32, (bm, n_seq), 1)
    posk = jnp.min(jnp.where(tok == _PLACEHOLDER, iota, n_seq + 1),
                   axis=1, keepdims=True)
    j = lax.broadcasted_iota(jnp.int32, (bm, 1), 0) & 31
    rloc_ref[...] = j * n_seq + posk


def _mlp(tokenized_text, face_img_embeddings, W0, b0, W1, b1,
         celeb_mean, celeb_std, batch, n_seq, token_dim):
    dim_out = W0.shape[0]
    vit_dim = face_img_embeddings.shape[1]
    bmlp = 256
    return pl.pallas_call(
        _mlp_body,
        grid=(batch // bmlp,),
        in_specs=[
            pl.BlockSpec((bmlp, n_seq), lambda i: (i, 0)),
            pl.BlockSpec((bmlp, vit_dim), lambda i: (i, 0)),
            pl.BlockSpec((dim_out, vit_dim), lambda i: (0, 0)),
            pl.BlockSpec((1, dim_out), lambda i: (0, 0)),
            pl.BlockSpec((dim_out, dim_out), lambda i: (0, 0)),
            pl.BlockSpec((1, dim_out), lambda i: (0, 0)),
            pl.BlockSpec((2, token_dim), lambda i: (0, 0)),
            pl.BlockSpec((2, token_dim), lambda i: (0, 0)),
        ],
        out_specs=(
            pl.BlockSpec((bmlp, token_dim), lambda i: (i, 0)),
            pl.BlockSpec((bmlp, token_dim), lambda i: (i, 0)),
            pl.BlockSpec((bmlp, 1), lambda i: (i, 0)),
        ),
        out_shape=(
            jax.ShapeDtypeStruct((batch, token_dim), jnp.float32),
            jax.ShapeDtypeStruct((batch, token_dim), jnp.float32),
            jax.ShapeDtypeStruct((batch, 1), jnp.int32),
        ),
    )(tokenized_text, face_img_embeddings, W0, b0.reshape(1, dim_out), W1,
      b1.reshape(1, dim_out), celeb_mean, celeb_std)


def _merge_body(rloc_ref, emb_ref, tie0_ref, tie1_ref, out_ref):
    nrows = emb_ref.shape[0]
    bb = tie0_ref.shape[0]
    rl = rloc_ref[0]
    rl2 = lax.broadcast_in_dim(rl, (nrows, bb), (0, 1))
    riota = lax.broadcasted_iota(jnp.int32, (nrows, bb), 0)
    m0 = (riota == rl2).astype(jnp.float32)
    m1 = (riota == rl2 + 1).astype(jnp.float32)
    u = lax.dot_general(m0, tie0_ref[...], (((1,), (0,)), ((), ())),
                        preferred_element_type=jnp.float32)
    u = u + lax.dot_general(m1, tie1_ref[...], (((1,), (0,)), ((), ())),
                            preferred_element_type=jnp.float32)
    row_on = jnp.sum(m0 + m1, axis=1, keepdims=True)
    keep = 1.0 - lax.broadcast_in_dim(row_on, out_ref.shape, (0, 1))
    out_ref[...] = emb_ref[...] * keep + u


def kernel(tokenized_text, embedded_text, face_img_embeddings,
           W0, b0, W1, b1, celeb_mean, celeb_std):
    batch, n_seq, token_dim = embedded_text.shape

    tie0, tie1, rloc = _mlp(tokenized_text, face_img_embeddings,
                            W0, b0, W1, b1, celeb_mean, celeb_std,
                            batch, n_seq, token_dim)

    bb = 32
    nblk = batch // bb
    rloc3 = rloc.reshape(nblk, 1, bb)
    emb2d = embedded_text.reshape(batch * n_seq, token_dim)
    nrows = bb * n_seq
    out2d = pl.pallas_call(
        _merge_body,
        grid=(nblk,),
        in_specs=[
            pl.BlockSpec((1, 1, bb), lambda i: (i, 0, 0)),
            pl.BlockSpec((nrows, token_dim), lambda i: (i, 0)),
            pl.BlockSpec((bb, token_dim), lambda i: (i, 0)),
            pl.BlockSpec((bb, token_dim), lambda i: (i, 0)),
        ],
        out_specs=pl.BlockSpec((nrows, token_dim), lambda i: (i, 0)),
        out_shape=jax.ShapeDtypeStruct((batch * n_seq, token_dim), jnp.float32),
        compiler_params=pltpu.CompilerParams(
            dimension_semantics=("arbitrary",),
        ),
    )(rloc3, emb2d, tie0, tie1)
    return out2d.reshape(batch, n_seq, token_dim)
